# Initial kernel scaffold; baseline (speedup 1.0000x reference)
import jax
import jax.numpy as jnp
from jax.experimental import pallas as pl


def kernel(x, edge_index, W1, b1, W2, b2):
    raise NotImplementedError("write your pallas kernel here")


# speedup vs baseline: 24.6145x; 24.6145x over previous
import functools

import jax
import jax.numpy as jnp
from jax import lax
from jax.experimental import pallas as pl
from jax.experimental.pallas import tpu as pltpu
from jax.experimental.pallas import tpu_sc as plsc

N = 10000
NPAD = 10240
PAD_ROWS = NPAD - N
E = 320000
NC, NS = 2, 16
NW = NC * NS
BATCH = 128
NBATCH = 80
EPAD = NW * NBATCH * BATCH
ROWS2D = EPAD // BATCH
F = 128
C = 40
CPAD = 48

_mesh = plsc.VectorSubcoreMesh(
    core_axis_name="c", subcore_axis_name="s", num_cores=NC, num_subcores=NS)
_sc_params = pltpu.CompilerParams(use_tc_tiling_on_sc=False)


def _deg_body(src_hbm, dstd_hbm, w_hbm, out_hbm, idx_a, idx_b, w_v, zbuf,
              deg_sh):
    c = lax.axis_index("c")
    s = lax.axis_index("s")
    w = s * NC + c
    zero16 = jnp.zeros((16,), jnp.float32)

    @pl.loop(0, 80)
    def _zero(i):
        zbuf[pl.ds(i * 16, 16)] = zero16

    pltpu.sync_copy(zbuf, deg_sh.at[pl.ds(s * 1280, 1280)])
    pltpu.sync_copy(src_hbm.at[pl.ds(w * NBATCH, NBATCH)], idx_a)
    pltpu.sync_copy(dstd_hbm.at[pl.ds(w * NBATCH, NBATCH)], idx_b)
    pltpu.sync_copy(w_hbm.at[pl.ds(w * NBATCH, NBATCH)], w_v)
    plsc.subcore_barrier()

    @pl.loop(0, NBATCH)
    def _acc(j):
        pltpu.sync_copy(w_v.at[j], deg_sh.at[idx_a.at[j]], add=True)
        pltpu.sync_copy(w_v.at[j], deg_sh.at[idx_b.at[j]], add=True)

    plsc.subcore_barrier()
    sl = pl.ds(s * 1280, 1280)
    pltpu.sync_copy(deg_sh.at[sl], zbuf)
    pltpu.sync_copy(zbuf, out_hbm.at[pl.ds(c * (2 * NPAD) + s * 1280, 1280)])


_deg_call = pl.kernel(
    _deg_body,
    out_type=jax.ShapeDtypeStruct((2 * 2 * NPAD,), jnp.float32),
    mesh=_mesh,
    scratch_types=[
        pltpu.VMEM((NBATCH, BATCH), jnp.int32),
        pltpu.VMEM((NBATCH, BATCH), jnp.int32),
        pltpu.VMEM((NBATCH, BATCH), jnp.float32),
        pltpu.VMEM((1280,), jnp.float32),
        pltpu.VMEM_SHARED((2 * NPAD,), jnp.float32),
    ],
    compiler_params=_sc_params,
)


def _zero_and_load(D, src_hbm, dst_hbm, idx_s, idx_d, stage, acc_sh,
                   src_base, dst_base, nbatch, s):
    zero16 = jnp.zeros((16,), jnp.float32)

    @pl.loop(0, BATCH)
    def _zr(i):
        @pl.loop(0, D // 16)
        def _zc(j):
            stage[i, pl.ds(j * 16, 16)] = zero16

    for k in range(5):
        pltpu.sync_copy(stage, acc_sh.at[pl.ds(s * 640 + k * 128, 128)])
    pltpu.sync_copy(src_hbm.at[pl.ds(src_base, nbatch)], idx_s)
    pltpu.sync_copy(dst_hbm.at[pl.ds(dst_base, nbatch)], idx_d)
    plsc.subcore_barrier()


def _agg_loop(nbatch, h_hbm, idx_s, idx_d, rows0, rows1, sem0, sem1, acc_sh):
    pltpu.async_copy(h_hbm.at[idx_s.at[0]], rows0, sem0)
    pltpu.async_copy(h_hbm.at[idx_s.at[1]], rows1, sem1)

    @pl.loop(0, nbatch // 2 - 1)
    def _run(jj):
        j0 = jj * 2
        pltpu.make_async_copy(h_hbm.at[idx_s.at[j0]], rows0, sem0).wait()
        pltpu.sync_copy(rows0, acc_sh.at[idx_d.at[j0]], add=True)
        pltpu.async_copy(h_hbm.at[idx_s.at[j0 + 2]], rows0, sem0)
        pltpu.make_async_copy(h_hbm.at[idx_s.at[j0 + 1]], rows1, sem1).wait()
        pltpu.sync_copy(rows1, acc_sh.at[idx_d.at[j0 + 1]], add=True)
        pltpu.async_copy(h_hbm.at[idx_s.at[j0 + 3]], rows1, sem1)

    pltpu.make_async_copy(h_hbm.at[idx_s.at[nbatch - 2]], rows0, sem0).wait()
    pltpu.sync_copy(rows0, acc_sh.at[idx_d.at[nbatch - 2]], add=True)
    pltpu.make_async_copy(h_hbm.at[idx_s.at[nbatch - 1]], rows1, sem1).wait()
    pltpu.sync_copy(rows1, acc_sh.at[idx_d.at[nbatch - 1]], add=True)
    plsc.subcore_barrier()


def _write_out(D, out_hbm, stage, acc_sh, c, s):
    for k in range(5):
        sl = pl.ds(s * 640 + k * 128, 128)
        pltpu.sync_copy(acc_sh.at[sl], stage)
        pltpu.sync_copy(stage, out_hbm.at[pl.ds(c * NPAD + s * 640 + k * 128,
                                                128)])


NB1 = ROWS2D // NS


def _agg1_body(h_hbm, src_hbm, dst_hbm, out_hbm, idx_s, idx_d, rows0, rows1,
               stage, sem0, sem1, acc_sh):
    c = lax.axis_index("c")
    s = lax.axis_index("s")
    _zero_and_load(64, src_hbm, dst_hbm, idx_s, idx_d, stage, acc_sh,
                   c * ROWS2D + s * NB1, s * NB1, NB1, s)
    _agg_loop(NB1, h_hbm, idx_s, idx_d, rows0, rows1, sem0, sem1, acc_sh)
    _write_out(64, out_hbm, stage, acc_sh, c, s)


_agg1 = pl.kernel(
    _agg1_body,
    out_type=jax.ShapeDtypeStruct((2 * NPAD, 64), jnp.float32),
    mesh=_mesh,
    scratch_types=[
        pltpu.VMEM((NB1, BATCH), jnp.int32),
        pltpu.VMEM((NB1, BATCH), jnp.int32),
        pltpu.VMEM((BATCH, 64), jnp.float32),
        pltpu.VMEM((BATCH, 64), jnp.float32),
        pltpu.VMEM((BATCH, 64), jnp.float32),
        pltpu.SemaphoreType.DMA,
        pltpu.SemaphoreType.DMA,
        pltpu.VMEM_SHARED((NPAD, 64), jnp.float32),
    ],
    compiler_params=_sc_params,
)


def _agg2_body(h_hbm, src_hbm, dst_hbm, out_hbm, idx_s, idx_d, rows0, rows1,
               stage, sem0, sem1, acc_sh):
    c = lax.axis_index("c")
    s = lax.axis_index("s")
    w = s * NC + c
    _zero_and_load(CPAD, src_hbm, dst_hbm, idx_s, idx_d, stage, acc_sh,
                   w * NBATCH, w * NBATCH, NBATCH, s)
    _agg_loop(NBATCH, h_hbm, idx_s, idx_d, rows0, rows1, sem0, sem1, acc_sh)
    _write_out(CPAD, out_hbm, stage, acc_sh, c, s)


_agg2 = pl.kernel(
    _agg2_body,
    out_type=jax.ShapeDtypeStruct((2 * NPAD, CPAD), jnp.float32),
    mesh=_mesh,
    scratch_types=[
        pltpu.VMEM((NBATCH, BATCH), jnp.int32),
        pltpu.VMEM((NBATCH, BATCH), jnp.int32),
        pltpu.VMEM((BATCH, CPAD), jnp.float32),
        pltpu.VMEM((BATCH, CPAD), jnp.float32),
        pltpu.VMEM((BATCH, CPAD), jnp.float32),
        pltpu.SemaphoreType.DMA,
        pltpu.SemaphoreType.DMA,
        pltpu.VMEM_SHARED((NPAD, CPAD), jnp.float32),
    ],
    compiler_params=_sc_params,
)


_BLK = 512
_GRID = NPAD // _BLK


def _tc1_body(x_ref, w_ref, do_ref, h_ref):
    j = pl.program_id(0)
    d = do_ref[0, :] + do_ref[1, :]
    a = lax.rsqrt(jnp.maximum(d, 1.0))
    h = jnp.dot(x_ref[...], w_ref[...], preferred_element_type=jnp.float32)
    h_half = jnp.where(j == 0, h[:, :64], h[:, 64:])
    h_ref[...] = h_half * a[:, None]


def _tc1(x_pad, W1, deg_out_p):
    return pl.pallas_call(
        _tc1_body,
        grid=(2, _GRID),
        in_specs=[
            pl.BlockSpec((_BLK, F), lambda j, i: (i, 0)),
            pl.BlockSpec((F, F), lambda j, i: (0, 0)),
            pl.BlockSpec((2, _BLK), lambda j, i: (0, i)),
        ],
        out_specs=pl.BlockSpec((_BLK, 64), lambda j, i: (j * _GRID + i, 0)),
        out_shape=jax.ShapeDtypeStruct((2 * NPAD, 64), jnp.float32),
    )(x_pad, W1, deg_out_p)


def _tc2_body(p0_ref, p1_ref, di_ref, do_ref, b1_ref, w2_ref, out_ref):
    b = lax.rsqrt(jnp.maximum(di_ref[0, :] + di_ref[1, :], 1.0))
    a = lax.rsqrt(jnp.maximum(do_ref[0, :] + do_ref[1, :], 1.0))
    agg = jnp.concatenate([p0_ref[...], p1_ref[...]], axis=1)
    h = jnp.maximum(agg * b[:, None] + b1_ref[0, :][None, :], 0.0)
    hw = jnp.dot(h, w2_ref[...], preferred_element_type=jnp.float32)
    out_ref[...] = hw * a[:, None]


def _tc2(p0, p1, deg_in_p, deg_out_p, b1_2d, W2p):
    return pl.pallas_call(
        _tc2_body,
        grid=(_GRID,),
        in_specs=[
            pl.BlockSpec((_BLK, 64), lambda i: (i, 0)),
            pl.BlockSpec((_BLK, 64), lambda i: (i, 0)),
            pl.BlockSpec((2, _BLK), lambda i: (0, i)),
            pl.BlockSpec((2, _BLK), lambda i: (0, i)),
            pl.BlockSpec((1, F), lambda i: (0, 0)),
            pl.BlockSpec((F, CPAD), lambda i: (0, 0)),
        ],
        out_specs=pl.BlockSpec((_BLK, CPAD), lambda i: (i, 0)),
        out_shape=jax.ShapeDtypeStruct((NPAD, CPAD), jnp.float32),
    )(p0, p1, deg_in_p, deg_out_p, b1_2d, W2p)


def _tc3_body(p0_ref, p1_ref, di_ref, b2_ref, out_ref):
    b = lax.rsqrt(jnp.maximum(di_ref[0, :] + di_ref[1, :], 1.0))
    agg = p0_ref[...] + p1_ref[...]
    out_ref[...] = agg * b[:, None] + b2_ref[0, :][None, :]


def _tc3(p0, p1, deg_in_p, b2_2d):
    return pl.pallas_call(
        _tc3_body,
        grid=(_GRID,),
        in_specs=[
            pl.BlockSpec((_BLK, CPAD), lambda i: (i, 0)),
            pl.BlockSpec((_BLK, CPAD), lambda i: (i, 0)),
            pl.BlockSpec((2, _BLK), lambda i: (0, i)),
            pl.BlockSpec((1, CPAD), lambda i: (0, 0)),
        ],
        out_specs=pl.BlockSpec((_BLK, CPAD), lambda i: (i, 0)),
        out_shape=jax.ShapeDtypeStruct((NPAD, CPAD), jnp.float32),
    )(p0, p1, deg_in_p, b2_2d)


def kernel(x, edge_index, W1, b1, W2, b2):
    src = edge_index[0]
    dst = edge_index[1]
    pad_i = jnp.arange(EPAD - E, dtype=jnp.int32)
    src_p = jnp.concatenate([src, pad_i % N]).reshape(ROWS2D, BATCH)
    dst_p = jnp.concatenate([dst, N + pad_i % PAD_ROWS]).reshape(ROWS2D, BATCH)
    dstd_p = dst_p + NPAD
    w_p = jnp.concatenate([
        jnp.ones((E,), jnp.float32),
        jnp.zeros((EPAD - E,), jnp.float32),
    ]).reshape(ROWS2D, BATCH)

    deg_flat = _deg_call(src_p, dstd_p, w_p)
    deg_out_p = jnp.stack(
        [deg_flat[0:NPAD], deg_flat[2 * NPAD:3 * NPAD]])
    deg_in_p = jnp.stack(
        [deg_flat[NPAD:2 * NPAD], deg_flat[3 * NPAD:4 * NPAD]])

    x_pad = jnp.pad(x, ((0, NPAD - N), (0, 0)))
    W2p = jnp.pad(W2, ((0, 0), (0, CPAD - C)))
    b1_2d = b1.reshape(1, F)
    b2_2d = jnp.pad(b2, (0, CPAD - C)).reshape(1, CPAD)

    src_cs = jnp.concatenate([src_p, src_p + NPAD])

    h1s = _tc1(x_pad, W1, deg_out_p)
    agg1 = _agg1(h1s, src_cs, dst_p)
    h2s = _tc2(agg1[:NPAD], agg1[NPAD:], deg_in_p, deg_out_p, b1_2d, W2p)
    agg2 = _agg2(h2s, src_p, dst_p)
    out = _tc3(agg2[:NPAD], agg2[NPAD:], deg_in_p, b2_2d)
    return out[:N, :C]


# speedup vs baseline: 27.1526x; 1.1031x over previous
import functools

import jax
import jax.numpy as jnp
from jax import lax
from jax.experimental import pallas as pl
from jax.experimental.pallas import tpu as pltpu
from jax.experimental.pallas import tpu_sc as plsc

N = 10000
NPAD = 10240
PAD_ROWS = NPAD - N
E = 320000
NC, NS = 2, 16
NW = NC * NS
BATCH = 128
NBATCH = 80
EPAD = NW * NBATCH * BATCH
ROWS2D = EPAD // BATCH
F = 128
C = 40
CPAD = 48

_mesh = plsc.VectorSubcoreMesh(
    core_axis_name="c", subcore_axis_name="s", num_cores=NC, num_subcores=NS)
_sc_params = pltpu.CompilerParams(use_tc_tiling_on_sc=False)


def _deg_body(src_hbm, dstd_hbm, w_hbm, out_hbm, idx_a, idx_b, w_v, zbuf,
              dsem, deg_sh):
    c = lax.axis_index("c")
    s = lax.axis_index("s")
    w = s * NC + c
    zero16 = jnp.zeros((16,), jnp.float32)

    @pl.loop(0, 80)
    def _zero(i):
        zbuf[pl.ds(i * 16, 16)] = zero16

    pltpu.sync_copy(zbuf, deg_sh.at[pl.ds(s * 1280, 1280)])
    pltpu.sync_copy(src_hbm.at[pl.ds(w * NBATCH, NBATCH)], idx_a)
    pltpu.sync_copy(dstd_hbm.at[pl.ds(w * NBATCH, NBATCH)], idx_b)
    pltpu.sync_copy(w_hbm.at[pl.ds(w * NBATCH, NBATCH)], w_v)
    plsc.subcore_barrier()

    @pl.loop(0, NBATCH)
    def _acc(j):
        pltpu.sync_copy(w_v.at[j], deg_sh.at[idx_a.at[j]], add=True)
        pltpu.sync_copy(w_v.at[j], deg_sh.at[idx_b.at[j]], add=True)

    plsc.subcore_barrier()
    sl = pl.ds(s * 1280, 1280)
    pltpu.sync_copy(deg_sh.at[sl], zbuf)
    pltpu.sync_copy(zbuf, out_hbm.at[pl.ds(c * (2 * NPAD) + s * 1280, 1280)])


_deg_call = pl.kernel(
    _deg_body,
    out_type=jax.ShapeDtypeStruct((2 * 2 * NPAD,), jnp.float32),
    mesh=_mesh,
    scratch_types=[
        pltpu.VMEM((NBATCH, BATCH), jnp.int32),
        pltpu.VMEM((NBATCH, BATCH), jnp.int32),
        pltpu.VMEM((NBATCH, BATCH), jnp.float32),
        pltpu.VMEM((1280,), jnp.float32),
        pltpu.SemaphoreType.DMA,
        pltpu.VMEM_SHARED((2 * NPAD,), jnp.float32),
    ],
    compiler_params=_sc_params,
)


def _zero_and_load(D, src_hbm, dst_hbm, idx_s, idx_d, stage, acc_sh,
                   src_base, dst_base, nbatch, s):
    zero16 = jnp.zeros((16,), jnp.float32)

    @pl.loop(0, BATCH)
    def _zr(i):
        @pl.loop(0, D // 16)
        def _zc(j):
            stage[i, pl.ds(j * 16, 16)] = zero16

    for k in range(5):
        pltpu.sync_copy(stage, acc_sh.at[pl.ds(s * 640 + k * 128, 128)])
    pltpu.sync_copy(src_hbm.at[pl.ds(src_base, nbatch)], idx_s)
    pltpu.sync_copy(dst_hbm.at[pl.ds(dst_base, nbatch)], idx_d)
    plsc.subcore_barrier()


_G = 2


def _agg_loop(nbatch, h_hbm, idx_s, idx_d, rows, gsems, ssems, acc_sh):
    ngrp = nbatch // _G

    def _gather(grp, bank):
        for b in range(_G):
            pltpu.async_copy(h_hbm.at[idx_s.at[grp * _G + b]],
                             rows.at[bank * _G + b], gsems[bank])

    def _drain_gather(grp, bank):
        for b in range(_G):
            pltpu.make_async_copy(h_hbm.at[idx_s.at[grp * _G + b]],
                                  rows.at[bank * _G + b], gsems[bank]).wait()

    def _scatter(grp, bank):
        for b in range(_G):
            pltpu.async_copy(rows.at[bank * _G + b],
                             acc_sh.at[idx_d.at[grp * _G + b]], ssems[bank],
                             add=True)

    def _drain_scatter(grp, bank):
        for b in range(_G):
            pltpu.make_async_copy(rows.at[bank * _G + b],
                                  acc_sh.at[idx_d.at[grp * _G + b]],
                                  ssems[bank]).wait()

    _gather(0, 0)
    _gather(1, 1)

    @pl.loop(0, (ngrp - 2) // 2)
    def _run(jj):
        g = jj * 2
        for bank in range(2):
            _drain_gather(g + bank, bank)
            _scatter(g + bank, bank)
            _drain_scatter(g + bank, bank)
            _gather(g + 2 + bank, bank)

    for bank in range(2):
        _drain_gather(ngrp - 2 + bank, bank)
        _scatter(ngrp - 2 + bank, bank)
        _drain_scatter(ngrp - 2 + bank, bank)
    plsc.subcore_barrier()


def _write_out(D, out_hbm, stage, acc_sh, c, s):
    for k in range(5):
        sl = pl.ds(s * 640 + k * 128, 128)
        pltpu.sync_copy(acc_sh.at[sl], stage)
        pltpu.sync_copy(stage, out_hbm.at[pl.ds(c * NPAD + s * 640 + k * 128,
                                                128)])


NB1 = ROWS2D // NS


def _agg1_body(h_hbm, src_hbm, dst_hbm, out_hbm, idx_s, idx_d, rows, stage,
               gsem0, gsem1, ssem0, ssem1, acc_sh):
    c = lax.axis_index("c")
    s = lax.axis_index("s")
    _zero_and_load(64, src_hbm, dst_hbm, idx_s, idx_d, stage, acc_sh,
                   c * ROWS2D + s * NB1, s * NB1, NB1, s)
    _agg_loop(NB1, h_hbm, idx_s, idx_d, rows, (gsem0, gsem1), (ssem0, ssem1),
              acc_sh)
    _write_out(64, out_hbm, stage, acc_sh, c, s)


_agg1 = pl.kernel(
    _agg1_body,
    out_type=jax.ShapeDtypeStruct((2 * NPAD, 64), jnp.float32),
    mesh=_mesh,
    scratch_types=[
        pltpu.VMEM((NB1, BATCH), jnp.int32),
        pltpu.VMEM((NB1, BATCH), jnp.int32),
        pltpu.VMEM((2 * _G, BATCH, 64), jnp.float32),
        pltpu.VMEM((BATCH, 64), jnp.float32),
        pltpu.SemaphoreType.DMA,
        pltpu.SemaphoreType.DMA,
        pltpu.SemaphoreType.DMA,
        pltpu.SemaphoreType.DMA,
        pltpu.VMEM_SHARED((NPAD, 64), jnp.float32),
    ],
    compiler_params=_sc_params,
)


def _agg2_body(h_hbm, src_hbm, dst_hbm, out_hbm, idx_s, idx_d, rows, stage,
               gsem0, gsem1, ssem0, ssem1, acc_sh):
    c = lax.axis_index("c")
    s = lax.axis_index("s")
    w = s * NC + c
    _zero_and_load(CPAD, src_hbm, dst_hbm, idx_s, idx_d, stage, acc_sh,
                   w * NBATCH, w * NBATCH, NBATCH, s)
    _agg_loop(NBATCH, h_hbm, idx_s, idx_d, rows, (gsem0, gsem1),
              (ssem0, ssem1), acc_sh)
    _write_out(CPAD, out_hbm, stage, acc_sh, c, s)


_agg2 = pl.kernel(
    _agg2_body,
    out_type=jax.ShapeDtypeStruct((2 * NPAD, CPAD), jnp.float32),
    mesh=_mesh,
    scratch_types=[
        pltpu.VMEM((NBATCH, BATCH), jnp.int32),
        pltpu.VMEM((NBATCH, BATCH), jnp.int32),
        pltpu.VMEM((2 * _G, BATCH, CPAD), jnp.float32),
        pltpu.VMEM((BATCH, CPAD), jnp.float32),
        pltpu.SemaphoreType.DMA,
        pltpu.SemaphoreType.DMA,
        pltpu.SemaphoreType.DMA,
        pltpu.SemaphoreType.DMA,
        pltpu.VMEM_SHARED((NPAD, CPAD), jnp.float32),
    ],
    compiler_params=_sc_params,
)


_BLK = 512
_GRID = NPAD // _BLK


def _tc1_body(x_ref, w_ref, do_ref, h_ref):
    j = pl.program_id(0)
    d = do_ref[0, :] + do_ref[1, :]
    a = lax.rsqrt(jnp.maximum(d, 1.0))
    h = jnp.dot(x_ref[...], w_ref[...], preferred_element_type=jnp.float32)
    h_half = jnp.where(j == 0, h[:, :64], h[:, 64:])
    h_ref[...] = h_half * a[:, None]


def _tc1(x_pad, W1, deg_out_p):
    return pl.pallas_call(
        _tc1_body,
        grid=(2, _GRID),
        in_specs=[
            pl.BlockSpec((_BLK, F), lambda j, i: (i, 0)),
            pl.BlockSpec((F, F), lambda j, i: (0, 0)),
            pl.BlockSpec((2, _BLK), lambda j, i: (0, i)),
        ],
        out_specs=pl.BlockSpec((_BLK, 64), lambda j, i: (j * _GRID + i, 0)),
        out_shape=jax.ShapeDtypeStruct((2 * NPAD, 64), jnp.float32),
    )(x_pad, W1, deg_out_p)


def _tc2_body(p0_ref, p1_ref, di_ref, do_ref, b1_ref, w2_ref, out_ref):
    b = lax.rsqrt(jnp.maximum(di_ref[0, :] + di_ref[1, :], 1.0))
    a = lax.rsqrt(jnp.maximum(do_ref[0, :] + do_ref[1, :], 1.0))
    agg = jnp.concatenate([p0_ref[...], p1_ref[...]], axis=1)
    h = jnp.maximum(agg * b[:, None] + b1_ref[0, :][None, :], 0.0)
    hw = jnp.dot(h, w2_ref[...], preferred_element_type=jnp.float32)
    out_ref[...] = hw * a[:, None]


def _tc2(p0, p1, deg_in_p, deg_out_p, b1_2d, W2p):
    return pl.pallas_call(
        _tc2_body,
        grid=(_GRID,),
        in_specs=[
            pl.BlockSpec((_BLK, 64), lambda i: (i, 0)),
            pl.BlockSpec((_BLK, 64), lambda i: (i, 0)),
            pl.BlockSpec((2, _BLK), lambda i: (0, i)),
            pl.BlockSpec((2, _BLK), lambda i: (0, i)),
            pl.BlockSpec((1, F), lambda i: (0, 0)),
            pl.BlockSpec((F, CPAD), lambda i: (0, 0)),
        ],
        out_specs=pl.BlockSpec((_BLK, CPAD), lambda i: (i, 0)),
        out_shape=jax.ShapeDtypeStruct((NPAD, CPAD), jnp.float32),
    )(p0, p1, deg_in_p, deg_out_p, b1_2d, W2p)


def _tc3_body(p0_ref, p1_ref, di_ref, b2_ref, out_ref):
    b = lax.rsqrt(jnp.maximum(di_ref[0, :] + di_ref[1, :], 1.0))
    agg = p0_ref[...] + p1_ref[...]
    out_ref[...] = agg * b[:, None] + b2_ref[0, :][None, :]


def _tc3(p0, p1, deg_in_p, b2_2d):
    return pl.pallas_call(
        _tc3_body,
        grid=(_GRID,),
        in_specs=[
            pl.BlockSpec((_BLK, CPAD), lambda i: (i, 0)),
            pl.BlockSpec((_BLK, CPAD), lambda i: (i, 0)),
            pl.BlockSpec((2, _BLK), lambda i: (0, i)),
            pl.BlockSpec((1, CPAD), lambda i: (0, 0)),
        ],
        out_specs=pl.BlockSpec((_BLK, CPAD), lambda i: (i, 0)),
        out_shape=jax.ShapeDtypeStruct((NPAD, CPAD), jnp.float32),
    )(p0, p1, deg_in_p, b2_2d)


def kernel(x, edge_index, W1, b1, W2, b2):
    src = edge_index[0]
    dst = edge_index[1]
    pad_i = jnp.arange(EPAD - E, dtype=jnp.int32)
    src_p = jnp.concatenate([src, pad_i % N]).reshape(ROWS2D, BATCH)
    dst_p = jnp.concatenate([dst, N + pad_i % PAD_ROWS]).reshape(ROWS2D, BATCH)
    dstd_p = dst_p + NPAD
    w_p = jnp.concatenate([
        jnp.ones((E,), jnp.float32),
        jnp.zeros((EPAD - E,), jnp.float32),
    ]).reshape(ROWS2D, BATCH)

    deg_flat = _deg_call(src_p, dstd_p, w_p)
    deg_out_p = jnp.stack(
        [deg_flat[0:NPAD], deg_flat[2 * NPAD:3 * NPAD]])
    deg_in_p = jnp.stack(
        [deg_flat[NPAD:2 * NPAD], deg_flat[3 * NPAD:4 * NPAD]])

    x_pad = jnp.pad(x, ((0, NPAD - N), (0, 0)))
    W2p = jnp.pad(W2, ((0, 0), (0, CPAD - C)))
    b1_2d = b1.reshape(1, F)
    b2_2d = jnp.pad(b2, (0, CPAD - C)).reshape(1, CPAD)

    src_cs = jnp.concatenate([src_p, src_p + NPAD])

    h1s = _tc1(x_pad, W1, deg_out_p)
    agg1 = _agg1(h1s, src_cs, dst_p)
    h2s = _tc2(agg1[:NPAD], agg1[NPAD:], deg_in_p, deg_out_p, b1_2d, W2p)
    agg2 = _agg2(h2s, src_p, dst_p)
    out = _tc3(agg2[:NPAD], agg2[NPAD:], deg_in_p, b2_2d)
    return out[:N, :C]


# speedup vs baseline: 29.3839x; 1.0822x over previous
import jax
import jax.numpy as jnp
from jax import lax
from jax.experimental import pallas as pl
from jax.experimental.pallas import tpu as pltpu
from jax.experimental.pallas import tpu_sc as plsc

N = 10000
NPAD = 10240
PAD_ROWS = NPAD - N
E = 320000
NC, NS = 2, 16
NW = NC * NS
BATCH = 128
NBATCH = 80
EPAD = NW * NBATCH * BATCH
ROWS2D = EPAD // BATCH
NB1 = ROWS2D // NS
F = 128
C = 40
CPAD = 48

_mesh = plsc.VectorSubcoreMesh(
    core_axis_name="c", subcore_axis_name="s", num_cores=NC, num_subcores=NS)
_sc_params = pltpu.CompilerParams(use_tc_tiling_on_sc=False)


def _deg_body(src_hbm, dst_hbm, out_hbm, idx_a, idx_b, ones_v, zbuf, deg_sh):
    c = lax.axis_index("c")
    s = lax.axis_index("s")
    w = s * NC + c
    zero16 = jnp.zeros((16,), jnp.float32)
    one16 = jnp.ones((16,), jnp.float32)

    @pl.loop(0, 80)
    def _zero(i):
        zbuf[pl.ds(i * 16, 16)] = zero16

    @pl.loop(0, BATCH // 16)
    def _ones(i):
        ones_v[pl.ds(i * 16, 16)] = one16

    pltpu.sync_copy(zbuf, deg_sh.at[pl.ds(s * 1280, 1280)])
    pltpu.sync_copy(src_hbm.at[pl.ds(w * NBATCH, NBATCH)], idx_a)
    pltpu.sync_copy(dst_hbm.at[pl.ds(w * NBATCH, NBATCH)], idx_b)

    @pl.loop(0, NBATCH)
    def _shift(j):
        for k in range(BATCH // 16):
            sl = pl.ds(k * 16, 16)
            idx_b[j, sl] = idx_b[j, sl] + NPAD

    plsc.subcore_barrier()

    @pl.loop(0, NBATCH)
    def _acc(j):
        pltpu.sync_copy(ones_v, deg_sh.at[idx_a.at[j]], add=True)
        pltpu.sync_copy(ones_v, deg_sh.at[idx_b.at[j]], add=True)

    plsc.subcore_barrier()
    sl = pl.ds(s * 1280, 1280)
    pltpu.sync_copy(deg_sh.at[sl], zbuf)
    pltpu.sync_copy(zbuf, out_hbm.at[pl.ds(c * (2 * NPAD) + s * 1280, 1280)])


_deg_call = pl.kernel(
    _deg_body,
    out_type=jax.ShapeDtypeStruct((2 * 2 * NPAD,), jnp.float32),
    mesh=_mesh,
    scratch_types=[
        pltpu.VMEM((NBATCH, BATCH), jnp.int32),
        pltpu.VMEM((NBATCH, BATCH), jnp.int32),
        pltpu.VMEM((BATCH,), jnp.float32),
        pltpu.VMEM((1280,), jnp.float32),
        pltpu.VMEM_SHARED((2 * NPAD,), jnp.float32),
    ],
    compiler_params=_sc_params,
)


def _zero_and_load(D, src_hbm, dst_hbm, idx_s, idx_d, stage, acc_sh,
                   src_base, dst_base, nbatch, s):
    zero16 = jnp.zeros((16,), jnp.float32)

    @pl.loop(0, BATCH)
    def _zr(i):
        @pl.loop(0, D // 16)
        def _zc(j):
            stage[i, pl.ds(j * 16, 16)] = zero16

    for k in range(5):
        pltpu.sync_copy(stage, acc_sh.at[pl.ds(s * 640 + k * 128, 128)])
    pltpu.sync_copy(src_hbm.at[pl.ds(src_base, nbatch)], idx_s)
    pltpu.sync_copy(dst_hbm.at[pl.ds(dst_base, nbatch)], idx_d)


_G = 2


def _agg_loop(nbatch, h_hbm, idx_s, idx_d, rows, gsems, ssems, acc_sh):
    ngrp = nbatch // _G

    def _gather(grp, bank):
        for b in range(_G):
            pltpu.async_copy(h_hbm.at[idx_s.at[grp * _G + b]],
                             rows.at[bank * _G + b], gsems[bank])

    def _drain_gather(grp, bank):
        for b in range(_G):
            pltpu.make_async_copy(h_hbm.at[idx_s.at[grp * _G + b]],
                                  rows.at[bank * _G + b], gsems[bank]).wait()

    def _scatter(grp, bank):
        for b in range(_G):
            pltpu.async_copy(rows.at[bank * _G + b],
                             acc_sh.at[idx_d.at[grp * _G + b]], ssems[bank],
                             add=True)

    def _drain_scatter(grp, bank):
        for b in range(_G):
            pltpu.make_async_copy(rows.at[bank * _G + b],
                                  acc_sh.at[idx_d.at[grp * _G + b]],
                                  ssems[bank]).wait()

    _gather(0, 0)
    _gather(1, 1)

    @pl.loop(0, (ngrp - 2) // 2)
    def _run(jj):
        g = jj * 2
        for bank in range(2):
            _drain_gather(g + bank, bank)
            _scatter(g + bank, bank)
            _drain_scatter(g + bank, bank)
            _gather(g + 2 + bank, bank)

    for bank in range(2):
        _drain_gather(ngrp - 2 + bank, bank)
        _scatter(ngrp - 2 + bank, bank)
        _drain_scatter(ngrp - 2 + bank, bank)
    plsc.subcore_barrier()


def _write_out(D, out_hbm, stage, acc_sh, c, s):
    for k in range(5):
        sl = pl.ds(s * 640 + k * 128, 128)
        pltpu.sync_copy(acc_sh.at[sl], stage)
        pltpu.sync_copy(stage, out_hbm.at[pl.ds(c * NPAD + s * 640 + k * 128,
                                                128)])


def _agg1_body(h_hbm, src_hbm, dst_hbm, out_hbm, idx_s, idx_d, rows, stage,
               gsem0, gsem1, ssem0, ssem1, acc_sh):
    c = lax.axis_index("c")
    s = lax.axis_index("s")
    _zero_and_load(64, src_hbm, dst_hbm, idx_s, idx_d, stage, acc_sh,
                   s * NB1, s * NB1, NB1, s)
    off = c * NPAD

    @pl.loop(0, NB1)
    def _shift(j):
        for k in range(BATCH // 16):
            sl = pl.ds(k * 16, 16)
            idx_s[j, sl] = idx_s[j, sl] + off

    plsc.subcore_barrier()
    _agg_loop(NB1, h_hbm, idx_s, idx_d, rows, (gsem0, gsem1), (ssem0, ssem1),
              acc_sh)
    _write_out(64, out_hbm, stage, acc_sh, c, s)


_agg1 = pl.kernel(
    _agg1_body,
    out_type=jax.ShapeDtypeStruct((2 * NPAD, 64), jnp.float32),
    mesh=_mesh,
    scratch_types=[
        pltpu.VMEM((NB1, BATCH), jnp.int32),
        pltpu.VMEM((NB1, BATCH), jnp.int32),
        pltpu.VMEM((2 * _G, BATCH, 64), jnp.float32),
        pltpu.VMEM((BATCH, 64), jnp.float32),
        pltpu.SemaphoreType.DMA,
        pltpu.SemaphoreType.DMA,
        pltpu.SemaphoreType.DMA,
        pltpu.SemaphoreType.DMA,
        pltpu.VMEM_SHARED((NPAD, 64), jnp.float32),
    ],
    compiler_params=_sc_params,
)


def _agg2_body(h_hbm, src_hbm, dst_hbm, out_hbm, idx_s, idx_d, rows, stage,
               gsem0, gsem1, ssem0, ssem1, acc_sh):
    c = lax.axis_index("c")
    s = lax.axis_index("s")
    w = s * NC + c
    _zero_and_load(CPAD, src_hbm, dst_hbm, idx_s, idx_d, stage, acc_sh,
                   w * NBATCH, w * NBATCH, NBATCH, s)
    plsc.subcore_barrier()
    _agg_loop(NBATCH, h_hbm, idx_s, idx_d, rows, (gsem0, gsem1),
              (ssem0, ssem1), acc_sh)
    _write_out(CPAD, out_hbm, stage, acc_sh, c, s)


_agg2 = pl.kernel(
    _agg2_body,
    out_type=jax.ShapeDtypeStruct((2 * NPAD, CPAD), jnp.float32),
    mesh=_mesh,
    scratch_types=[
        pltpu.VMEM((NBATCH, BATCH), jnp.int32),
        pltpu.VMEM((NBATCH, BATCH), jnp.int32),
        pltpu.VMEM((2 * _G, BATCH, CPAD), jnp.float32),
        pltpu.VMEM((BATCH, CPAD), jnp.float32),
        pltpu.SemaphoreType.DMA,
        pltpu.SemaphoreType.DMA,
        pltpu.SemaphoreType.DMA,
        pltpu.SemaphoreType.DMA,
        pltpu.VMEM_SHARED((NPAD, CPAD), jnp.float32),
    ],
    compiler_params=_sc_params,
)


_BLK = 1024
_GRID = NPAD // _BLK


def _tc1_body(x_ref, w_ref, dg_ref, h_ref):
    j = pl.program_id(0)
    a = lax.rsqrt(jnp.maximum(dg_ref[0, :] + dg_ref[2, :], 1.0))
    h = jnp.dot(x_ref[...], w_ref[...], preferred_element_type=jnp.float32)
    h_half = jnp.where(j == 0, h[:, :64], h[:, 64:])
    h_ref[...] = h_half * a[:, None]


def _tc1(x, W1, deg4):
    return pl.pallas_call(
        _tc1_body,
        grid=(2, _GRID),
        in_specs=[
            pl.BlockSpec((_BLK, F), lambda j, i: (i, 0)),
            pl.BlockSpec((F, F), lambda j, i: (0, 0)),
            pl.BlockSpec((4, _BLK), lambda j, i: (0, i)),
        ],
        out_specs=pl.BlockSpec((_BLK, 64), lambda j, i: (j * _GRID + i, 0)),
        out_shape=jax.ShapeDtypeStruct((2 * NPAD, 64), jnp.float32),
    )(x, W1, deg4)


def _tc2_body(p0_ref, p1_ref, dg_ref, b1_ref, w2_ref, out_ref):
    a = lax.rsqrt(jnp.maximum(dg_ref[0, :] + dg_ref[2, :], 1.0))
    b = lax.rsqrt(jnp.maximum(dg_ref[1, :] + dg_ref[3, :], 1.0))
    agg = jnp.concatenate([p0_ref[...], p1_ref[...]], axis=1)
    h = jnp.maximum(agg * b[:, None] + b1_ref[0, :][None, :], 0.0)
    hw = jnp.dot(h, w2_ref[...], preferred_element_type=jnp.float32)
    out_ref[...] = hw * a[:, None]


def _tc2(p0, p1, deg4, b1_2d, W2p):
    return pl.pallas_call(
        _tc2_body,
        grid=(_GRID,),
        in_specs=[
            pl.BlockSpec((_BLK, 64), lambda i: (i, 0)),
            pl.BlockSpec((_BLK, 64), lambda i: (i, 0)),
            pl.BlockSpec((4, _BLK), lambda i: (0, i)),
            pl.BlockSpec((1, F), lambda i: (0, 0)),
            pl.BlockSpec((F, CPAD), lambda i: (0, 0)),
        ],
        out_specs=pl.BlockSpec((_BLK, CPAD), lambda i: (i, 0)),
        out_shape=jax.ShapeDtypeStruct((NPAD, CPAD), jnp.float32),
    )(p0, p1, deg4, b1_2d, W2p)


_BLK3 = 1024


def _tc3_body(p0_ref, p1_ref, dg_ref, b2_ref, out_ref):
    b = lax.rsqrt(jnp.maximum(dg_ref[1, :] + dg_ref[3, :], 1.0))
    agg = p0_ref[...] + p1_ref[...]
    out_ref[...] = agg[:, :C] * b[:, None] + b2_ref[0, :][None, :]


def _tc3(p0, p1, deg4, b2_2d):
    return pl.pallas_call(
        _tc3_body,
        grid=(pl.cdiv(N, _BLK3),),
        in_specs=[
            pl.BlockSpec((_BLK3, CPAD), lambda i: (i, 0)),
            pl.BlockSpec((_BLK3, CPAD), lambda i: (i, 0)),
            pl.BlockSpec((4, _BLK3), lambda i: (0, i)),
            pl.BlockSpec((1, C), lambda i: (0, 0)),
        ],
        out_specs=pl.BlockSpec((_BLK3, C), lambda i: (i, 0)),
        out_shape=jax.ShapeDtypeStruct((N, C), jnp.float32),
    )(p0, p1, deg4, b2_2d)


def kernel(x, edge_index, W1, b1, W2, b2):
    src = edge_index[0]
    dst = edge_index[1]
    pad_i = N + (jnp.arange(EPAD - E, dtype=jnp.int32) % PAD_ROWS)
    src_p = jnp.concatenate([src, pad_i]).reshape(ROWS2D, BATCH)
    dst_p = jnp.concatenate([dst, pad_i]).reshape(ROWS2D, BATCH)

    deg4 = _deg_call(src_p, dst_p).reshape(4, NPAD)

    W2p = jnp.pad(W2, ((0, 0), (0, CPAD - C)))
    b1_2d = b1.reshape(1, F)
    b2_2d = b2.reshape(1, C)

    h1s = _tc1(x, W1, deg4)
    agg1 = _agg1(h1s, src_p, dst_p)
    h2s = _tc2(agg1[:NPAD], agg1[NPAD:], deg4, b1_2d, W2p)
    agg2 = _agg2(h2s, src_p, dst_p)
    return _tc3(agg2[:NPAD], agg2[NPAD:], deg4, b2_2d)


# speedup vs baseline: 34.3512x; 1.1690x over previous
import jax
import jax.numpy as jnp
from jax import lax
from jax.experimental import pallas as pl
from jax.experimental.pallas import tpu as pltpu
from jax.experimental.pallas import tpu_sc as plsc

N = 10000
NPAD = 10240
PAD_ROWS = NPAD - N
E = 320000
NC, NS = 2, 16
NW = NC * NS
BATCH = 128
NBATCH = 80
EPAD = NW * NBATCH * BATCH
ROWS2D = EPAD // BATCH
NB1 = ROWS2D // NS
F = 128
C = 40
CPAD = 64

_mesh = plsc.VectorSubcoreMesh(
    core_axis_name="c", subcore_axis_name="s", num_cores=NC, num_subcores=NS)
_sc_params = pltpu.CompilerParams(use_tc_tiling_on_sc=False)


def _deg_body(src_hbm, dst_hbm, out_hbm, idx_a, idx_b, ones_v, zbuf, deg_sh):
    c = lax.axis_index("c")
    s = lax.axis_index("s")
    w = s * NC + c
    zero16 = jnp.zeros((16,), jnp.float32)
    one16 = jnp.ones((16,), jnp.float32)

    @pl.loop(0, 80)
    def _zero(i):
        zbuf[pl.ds(i * 16, 16)] = zero16

    @pl.loop(0, BATCH // 16)
    def _ones(i):
        ones_v[pl.ds(i * 16, 16)] = one16

    pltpu.sync_copy(zbuf, deg_sh.at[pl.ds(s * 1280, 1280)])
    pltpu.sync_copy(src_hbm.at[pl.ds(w * NBATCH, NBATCH)], idx_a)
    pltpu.sync_copy(dst_hbm.at[pl.ds(w * NBATCH, NBATCH)], idx_b)

    @pl.loop(0, NBATCH)
    def _shift(j):
        for k in range(BATCH // 16):
            sl = pl.ds(k * 16, 16)
            idx_b[j, sl] = idx_b[j, sl] + NPAD

    plsc.subcore_barrier()

    @pl.loop(0, NBATCH)
    def _acc(j):
        pltpu.sync_copy(ones_v, deg_sh.at[idx_a.at[j]], add=True)
        pltpu.sync_copy(ones_v, deg_sh.at[idx_b.at[j]], add=True)

    plsc.subcore_barrier()
    sl = pl.ds(s * 1280, 1280)
    pltpu.sync_copy(deg_sh.at[sl], zbuf)
    pltpu.sync_copy(zbuf, out_hbm.at[pl.ds(c * (2 * NPAD) + s * 1280, 1280)])


_deg_call = pl.kernel(
    _deg_body,
    out_type=jax.ShapeDtypeStruct((2 * 2 * NPAD,), jnp.float32),
    mesh=_mesh,
    scratch_types=[
        pltpu.VMEM((NBATCH, BATCH), jnp.int32),
        pltpu.VMEM((NBATCH, BATCH), jnp.int32),
        pltpu.VMEM((BATCH,), jnp.float32),
        pltpu.VMEM((1280,), jnp.float32),
        pltpu.VMEM_SHARED((2 * NPAD,), jnp.float32),
    ],
    compiler_params=_sc_params,
)


_G = 2


def _agg_loop(nbatch, h_hbm, idx_s, idx_d, rows, gsems, ssems, acc_sh):
    ngrp = nbatch // _G

    def _gather(grp, bank):
        for b in range(_G):
            pltpu.async_copy(h_hbm.at[idx_s.at[grp * _G + b]],
                             rows.at[bank * _G + b], gsems[bank])

    def _drain_gather(grp, bank):
        for b in range(_G):
            pltpu.make_async_copy(h_hbm.at[idx_s.at[grp * _G + b]],
                                  rows.at[bank * _G + b], gsems[bank]).wait()

    def _scatter(grp, bank):
        for b in range(_G):
            pltpu.async_copy(rows.at[bank * _G + b],
                             acc_sh.at[idx_d.at[grp * _G + b]], ssems[bank],
                             add=True)

    def _drain_scatter(grp, bank):
        for b in range(_G):
            pltpu.make_async_copy(rows.at[bank * _G + b],
                                  acc_sh.at[idx_d.at[grp * _G + b]],
                                  ssems[bank]).wait()

    _gather(0, 0)
    _gather(1, 1)

    @pl.loop(0, (ngrp - 2) // 2)
    def _run(jj):
        g = jj * 2
        for bank in range(2):
            _drain_gather(g + bank, bank)
            _scatter(g + bank, bank)
            _drain_scatter(g + bank, bank)
            _gather(g + 2 + bank, bank)

    for bank in range(2):
        _drain_gather(ngrp - 2 + bank, bank)
        _scatter(ngrp - 2 + bank, bank)
        _drain_scatter(ngrp - 2 + bank, bank)
    plsc.subcore_barrier()


def _make_agg_body(D, mul):
    def _body(h_hbm, src_hbm, dst_hbm, out_hbm, idx_s, idx_d, rows, stage,
              gsem0, gsem1, ssem0, ssem1, acc_sh):
        c = lax.axis_index("c")
        s = lax.axis_index("s")
        zero16 = jnp.zeros((16,), jnp.float32)

        @pl.loop(0, BATCH)
        def _zr(i):
            @pl.loop(0, D // 16)
            def _zc(j):
                stage[i, pl.ds(j * 16, 16)] = zero16

        for k in range(5):
            pltpu.sync_copy(stage, acc_sh.at[pl.ds(s * 640 + k * 128, 128)])
        pltpu.sync_copy(src_hbm.at[pl.ds(s * NB1, NB1)], idx_s)
        pltpu.sync_copy(dst_hbm.at[pl.ds(s * NB1, NB1)], idx_d)

        @pl.loop(0, NB1)
        def _shift(j):
            for k in range(BATCH // 16):
                sl = pl.ds(k * 16, 16)
                idx_s[j, sl] = idx_s[j, sl] * mul + c

        plsc.subcore_barrier()
        _agg_loop(NB1, h_hbm, idx_s, idx_d, rows, (gsem0, gsem1),
                  (ssem0, ssem1), acc_sh)
        for k in range(5):
            r0 = s * 640 + k * 128
            pltpu.sync_copy(acc_sh.at[pl.ds(r0, 128)], stage)
            pltpu.sync_copy(stage,
                            out_hbm.at[pl.ds(r0, 128), pl.ds(c * D, D)])

    return _body


def _make_agg(D, mul):
    return pl.kernel(
        _make_agg_body(D, mul),
        out_type=jax.ShapeDtypeStruct((NPAD, 128), jnp.float32),
        mesh=_mesh,
        scratch_types=[
            pltpu.VMEM((NB1, BATCH), jnp.int32),
            pltpu.VMEM((NB1, BATCH), jnp.int32),
            pltpu.VMEM((2 * _G, BATCH, D), jnp.float32),
            pltpu.VMEM((BATCH, D), jnp.float32),
            pltpu.SemaphoreType.DMA,
            pltpu.SemaphoreType.DMA,
            pltpu.SemaphoreType.DMA,
            pltpu.SemaphoreType.DMA,
            pltpu.VMEM_SHARED((NPAD, D), jnp.float32),
        ],
        compiler_params=_sc_params,
    )


_agg1 = _make_agg(64, 2)
_agg2 = _make_agg(32, 4)


_BLK = 1024
_GRID = NPAD // _BLK


def _tc1_body(x_ref, w_ref, dg_ref, h_ref):
    a = lax.rsqrt(jnp.maximum(dg_ref[0, :] + dg_ref[2, :], 1.0))
    h = jnp.dot(x_ref[...], w_ref[...], preferred_element_type=jnp.float32)
    h_ref[...] = h * a[:, None]


def _tc1(x, W1, deg4):
    return pl.pallas_call(
        _tc1_body,
        grid=(_GRID,),
        in_specs=[
            pl.BlockSpec((_BLK, F), lambda i: (i, 0)),
            pl.BlockSpec((F, F), lambda i: (0, 0)),
            pl.BlockSpec((4, _BLK), lambda i: (0, i)),
        ],
        out_specs=pl.BlockSpec((_BLK, F), lambda i: (i, 0)),
        out_shape=jax.ShapeDtypeStruct((NPAD, F), jnp.float32),
    )(x, W1, deg4)


def _tc2_body(p_ref, dg_ref, b1_ref, w2_ref, out_ref):
    a = lax.rsqrt(jnp.maximum(dg_ref[0, :] + dg_ref[2, :], 1.0))
    b = lax.rsqrt(jnp.maximum(dg_ref[1, :] + dg_ref[3, :], 1.0))
    h = jnp.maximum(p_ref[...] * b[:, None] + b1_ref[0, :][None, :], 0.0)
    hw = jnp.dot(h, w2_ref[...], preferred_element_type=jnp.float32)
    hws = hw * a[:, None]
    out_ref[...] = jnp.concatenate(
        [hws, jnp.zeros((_BLK, 128 - CPAD), jnp.float32)], axis=1)


def _tc2(p, deg4, b1_2d, W2p):
    return pl.pallas_call(
        _tc2_body,
        grid=(_GRID,),
        in_specs=[
            pl.BlockSpec((_BLK, F), lambda i: (i, 0)),
            pl.BlockSpec((4, _BLK), lambda i: (0, i)),
            pl.BlockSpec((1, F), lambda i: (0, 0)),
            pl.BlockSpec((F, CPAD), lambda i: (0, 0)),
        ],
        out_specs=pl.BlockSpec((_BLK, F), lambda i: (i, 0)),
        out_shape=jax.ShapeDtypeStruct((NPAD, F), jnp.float32),
    )(p, deg4, b1_2d, W2p)


def _tc3_body(p_ref, dg_ref, b2_ref, out_ref):
    b = lax.rsqrt(jnp.maximum(dg_ref[1, :] + dg_ref[3, :], 1.0))
    out_ref[...] = p_ref[:, :C] * b[:, None] + b2_ref[0, :][None, :]


def _tc3(p, deg4, b2_2d):
    return pl.pallas_call(
        _tc3_body,
        grid=(pl.cdiv(N, _BLK),),
        in_specs=[
            pl.BlockSpec((_BLK, F), lambda i: (i, 0)),
            pl.BlockSpec((4, _BLK), lambda i: (0, i)),
            pl.BlockSpec((1, C), lambda i: (0, 0)),
        ],
        out_specs=pl.BlockSpec((_BLK, C), lambda i: (i, 0)),
        out_shape=jax.ShapeDtypeStruct((N, C), jnp.float32),
    )(p, deg4, b2_2d)


def kernel(x, edge_index, W1, b1, W2, b2):
    src = edge_index[0]
    dst = edge_index[1]
    pad_i = N + (jnp.arange(EPAD - E, dtype=jnp.int32) % PAD_ROWS)
    src_p = jnp.concatenate([src, pad_i]).reshape(ROWS2D, BATCH)
    dst_p = jnp.concatenate([dst, pad_i]).reshape(ROWS2D, BATCH)

    deg4 = _deg_call(src_p, dst_p).reshape(4, NPAD)

    W2p = jnp.pad(W2, ((0, 0), (0, CPAD - C)))
    b1_2d = b1.reshape(1, F)
    b2_2d = b2.reshape(1, C)

    h1s = _tc1(x, W1, deg4)
    agg1 = _agg1(h1s.reshape(2 * NPAD, 64), src_p, dst_p)
    h2s = _tc2(agg1, deg4, b1_2d, W2p)
    agg2 = _agg2(h2s.reshape(4 * NPAD, 32), src_p, dst_p)
    return _tc3(agg2, deg4, b2_2d)


# speedup vs baseline: 35.0044x; 1.0190x over previous
import jax
import jax.numpy as jnp
from jax import lax
from jax.experimental import pallas as pl
from jax.experimental.pallas import tpu as pltpu
from jax.experimental.pallas import tpu_sc as plsc

N = 10000
NPAD = 10240
PAD_ROWS = NPAD - N
E = 320000
NC, NS = 2, 16
NW = NC * NS
BATCH = 256
NBATCH = 40
EPAD = NW * NBATCH * BATCH
ROWS2D = EPAD // BATCH
NB1 = ROWS2D // NS
F = 128
C = 40
CPAD = 64

_mesh = plsc.VectorSubcoreMesh(
    core_axis_name="c", subcore_axis_name="s", num_cores=NC, num_subcores=NS)
_sc_params = pltpu.CompilerParams(use_tc_tiling_on_sc=False)


def _deg_body(src_hbm, dst_hbm, out_hbm, idx_a, idx_b, ones_v, zbuf, deg_sh):
    c = lax.axis_index("c")
    s = lax.axis_index("s")
    w = s * NC + c
    zero16 = jnp.zeros((16,), jnp.float32)
    one16 = jnp.ones((16,), jnp.float32)

    @pl.loop(0, 80)
    def _zero(i):
        zbuf[pl.ds(i * 16, 16)] = zero16

    @pl.loop(0, BATCH // 16)
    def _ones(i):
        ones_v[pl.ds(i * 16, 16)] = one16

    pltpu.sync_copy(zbuf, deg_sh.at[pl.ds(s * 1280, 1280)])
    pltpu.sync_copy(src_hbm.at[pl.ds(w * NBATCH, NBATCH)], idx_a)
    pltpu.sync_copy(dst_hbm.at[pl.ds(w * NBATCH, NBATCH)], idx_b)

    @pl.loop(0, NBATCH)
    def _shift(j):
        for k in range(BATCH // 16):
            sl = pl.ds(k * 16, 16)
            idx_b[j, sl] = idx_b[j, sl] + NPAD

    plsc.subcore_barrier()

    @pl.loop(0, NBATCH)
    def _acc(j):
        pltpu.sync_copy(ones_v, deg_sh.at[idx_a.at[j]], add=True)
        pltpu.sync_copy(ones_v, deg_sh.at[idx_b.at[j]], add=True)

    plsc.subcore_barrier()
    sl = pl.ds(s * 1280, 1280)
    pltpu.sync_copy(deg_sh.at[sl], zbuf)
    pltpu.sync_copy(zbuf, out_hbm.at[pl.ds(c * (2 * NPAD) + s * 1280, 1280)])


_deg_call = pl.kernel(
    _deg_body,
    out_type=jax.ShapeDtypeStruct((2 * 2 * NPAD,), jnp.float32),
    mesh=_mesh,
    scratch_types=[
        pltpu.VMEM((NBATCH, BATCH), jnp.int32),
        pltpu.VMEM((NBATCH, BATCH), jnp.int32),
        pltpu.VMEM((BATCH,), jnp.float32),
        pltpu.VMEM((1280,), jnp.float32),
        pltpu.VMEM_SHARED((2 * NPAD,), jnp.float32),
    ],
    compiler_params=_sc_params,
)


_G = 1


def _agg_loop(nbatch, h_hbm, idx_s, idx_d, rows, gsems, ssems, acc_sh):
    ngrp = nbatch // _G

    def _gather(grp, bank):
        for b in range(_G):
            pltpu.async_copy(h_hbm.at[idx_s.at[grp * _G + b]],
                             rows.at[bank * _G + b], gsems[bank])

    def _drain_gather(grp, bank):
        for b in range(_G):
            pltpu.make_async_copy(h_hbm.at[idx_s.at[grp * _G + b]],
                                  rows.at[bank * _G + b], gsems[bank]).wait()

    def _scatter(grp, bank):
        for b in range(_G):
            pltpu.async_copy(rows.at[bank * _G + b],
                             acc_sh.at[idx_d.at[grp * _G + b]], ssems[bank],
                             add=True)

    def _drain_scatter(grp, bank):
        for b in range(_G):
            pltpu.make_async_copy(rows.at[bank * _G + b],
                                  acc_sh.at[idx_d.at[grp * _G + b]],
                                  ssems[bank]).wait()

    _gather(0, 0)
    _gather(1, 1)

    @pl.loop(0, (ngrp - 2) // 2)
    def _run(jj):
        g = jj * 2
        for bank in range(2):
            _drain_gather(g + bank, bank)
            _scatter(g + bank, bank)
            _drain_scatter(g + bank, bank)
            _gather(g + 2 + bank, bank)

    for bank in range(2):
        _drain_gather(ngrp - 2 + bank, bank)
        _scatter(ngrp - 2 + bank, bank)
        _drain_scatter(ngrp - 2 + bank, bank)
    plsc.subcore_barrier()


def _make_agg_body(D, mul):
    def _body(h_hbm, src_hbm, dst_hbm, out_hbm, idx_s, idx_d, rows, stage,
              gsem0, gsem1, ssem0, ssem1, acc_sh):
        c = lax.axis_index("c")
        s = lax.axis_index("s")
        zero16 = jnp.zeros((16,), jnp.float32)

        @pl.loop(0, 128)
        def _zr(i):
            @pl.loop(0, D // 16)
            def _zc(j):
                stage[i, pl.ds(j * 16, 16)] = zero16

        for k in range(5):
            pltpu.sync_copy(stage, acc_sh.at[pl.ds(s * 640 + k * 128, 128)])
        pltpu.sync_copy(src_hbm.at[pl.ds(s * NB1, NB1)], idx_s)
        pltpu.sync_copy(dst_hbm.at[pl.ds(s * NB1, NB1)], idx_d)

        @pl.loop(0, NB1)
        def _shift(j):
            for k in range(BATCH // 16):
                sl = pl.ds(k * 16, 16)
                idx_s[j, sl] = idx_s[j, sl] * mul + c

        plsc.subcore_barrier()
        _agg_loop(NB1, h_hbm, idx_s, idx_d, rows, (gsem0, gsem1),
                  (ssem0, ssem1), acc_sh)
        for k in range(5):
            r0 = s * 640 + k * 128
            pltpu.sync_copy(acc_sh.at[pl.ds(r0, 128)], stage)
            pltpu.sync_copy(stage,
                            out_hbm.at[pl.ds(r0, 128), pl.ds(c * D, D)])

    return _body


def _make_agg(D, mul):
    return pl.kernel(
        _make_agg_body(D, mul),
        out_type=jax.ShapeDtypeStruct((NPAD, 128), jnp.float32),
        mesh=_mesh,
        scratch_types=[
            pltpu.VMEM((NB1, BATCH), jnp.int32),
            pltpu.VMEM((NB1, BATCH), jnp.int32),
            pltpu.VMEM((2 * _G, BATCH, D), jnp.float32),
            pltpu.VMEM((128, D), jnp.float32),
            pltpu.SemaphoreType.DMA,
            pltpu.SemaphoreType.DMA,
            pltpu.SemaphoreType.DMA,
            pltpu.SemaphoreType.DMA,
            pltpu.VMEM_SHARED((NPAD, D), jnp.float32),
        ],
        compiler_params=_sc_params,
    )


_agg1 = _make_agg(64, 2)
_agg2 = _make_agg(32, 4)


_BLK = 1024
_GRID = NPAD // _BLK


def _tc1_body(x_ref, w_ref, dg_ref, h_ref):
    a = lax.rsqrt(jnp.maximum(dg_ref[0, :] + dg_ref[2, :], 1.0))
    h = jnp.dot(x_ref[...], w_ref[...], preferred_element_type=jnp.float32)
    h_ref[...] = h * a[:, None]


def _tc1(x, W1, deg4):
    return pl.pallas_call(
        _tc1_body,
        grid=(_GRID,),
        in_specs=[
            pl.BlockSpec((_BLK, F), lambda i: (i, 0)),
            pl.BlockSpec((F, F), lambda i: (0, 0)),
            pl.BlockSpec((4, _BLK), lambda i: (0, i)),
        ],
        out_specs=pl.BlockSpec((_BLK, F), lambda i: (i, 0)),
        out_shape=jax.ShapeDtypeStruct((NPAD, F), jnp.float32),
    )(x, W1, deg4)


def _tc2_body(p_ref, dg_ref, b1_ref, w2_ref, out_ref):
    a = lax.rsqrt(jnp.maximum(dg_ref[0, :] + dg_ref[2, :], 1.0))
    b = lax.rsqrt(jnp.maximum(dg_ref[1, :] + dg_ref[3, :], 1.0))
    h = jnp.maximum(p_ref[...] * b[:, None] + b1_ref[0, :][None, :], 0.0)
    hw = jnp.dot(h, w2_ref[...], preferred_element_type=jnp.float32)
    hws = hw * a[:, None]
    out_ref[...] = jnp.concatenate(
        [hws, jnp.zeros((_BLK, 128 - CPAD), jnp.float32)], axis=1)


def _tc2(p, deg4, b1_2d, W2p):
    return pl.pallas_call(
        _tc2_body,
        grid=(_GRID,),
        in_specs=[
            pl.BlockSpec((_BLK, F), lambda i: (i, 0)),
            pl.BlockSpec((4, _BLK), lambda i: (0, i)),
            pl.BlockSpec((1, F), lambda i: (0, 0)),
            pl.BlockSpec((F, CPAD), lambda i: (0, 0)),
        ],
        out_specs=pl.BlockSpec((_BLK, F), lambda i: (i, 0)),
        out_shape=jax.ShapeDtypeStruct((NPAD, F), jnp.float32),
    )(p, deg4, b1_2d, W2p)


def _tc3_body(p_ref, dg_ref, b2_ref, out_ref):
    b = lax.rsqrt(jnp.maximum(dg_ref[1, :] + dg_ref[3, :], 1.0))
    out_ref[...] = p_ref[:, :C] * b[:, None] + b2_ref[0, :][None, :]


def _tc3(p, deg4, b2_2d):
    return pl.pallas_call(
        _tc3_body,
        grid=(pl.cdiv(N, _BLK),),
        in_specs=[
            pl.BlockSpec((_BLK, F), lambda i: (i, 0)),
            pl.BlockSpec((4, _BLK), lambda i: (0, i)),
            pl.BlockSpec((1, C), lambda i: (0, 0)),
        ],
        out_specs=pl.BlockSpec((_BLK, C), lambda i: (i, 0)),
        out_shape=jax.ShapeDtypeStruct((N, C), jnp.float32),
    )(p, deg4, b2_2d)


def kernel(x, edge_index, W1, b1, W2, b2):
    src = edge_index[0]
    dst = edge_index[1]
    pad_i = N + (jnp.arange(EPAD - E, dtype=jnp.int32) % PAD_ROWS)
    src_p = jnp.concatenate([src, pad_i]).reshape(ROWS2D, BATCH)
    dst_p = jnp.concatenate([dst, pad_i]).reshape(ROWS2D, BATCH)

    deg4 = _deg_call(src_p, dst_p).reshape(4, NPAD)

    W2p = jnp.pad(W2, ((0, 0), (0, CPAD - C)))
    b1_2d = b1.reshape(1, F)
    b2_2d = b2.reshape(1, C)

    h1s = _tc1(x, W1, deg4)
    agg1 = _agg1(h1s.reshape(2 * NPAD, 64), src_p, dst_p)
    h2s = _tc2(agg1, deg4, b1_2d, W2p)
    agg2 = _agg2(h2s.reshape(4 * NPAD, 32), src_p, dst_p)
    return _tc3(agg2, deg4, b2_2d)


# speedup vs baseline: 36.3243x; 1.0377x over previous
import jax
import jax.numpy as jnp
from jax import lax
from jax.experimental import pallas as pl
from jax.experimental.pallas import tpu as pltpu
from jax.experimental.pallas import tpu_sc as plsc

N = 10000
NPAD = 10240
PAD_ROWS = NPAD - N
E = 320000
NC, NS = 2, 16
NW = NC * NS
BATCH = 256
NBATCH = 40
EPAD = NW * NBATCH * BATCH
ROWS2D = EPAD // BATCH
NB1 = ROWS2D // NS
F = 128
C = 40
CPAD = 64

_mesh = plsc.VectorSubcoreMesh(
    core_axis_name="c", subcore_axis_name="s", num_cores=NC, num_subcores=NS)
_sc_params = pltpu.CompilerParams(use_tc_tiling_on_sc=False)


def _deg_body(src_hbm, dst_hbm, out_hbm, idx_a, idx_b, ones_v, zbuf, dsem0,
              dsem1, deg_sh):
    c = lax.axis_index("c")
    s = lax.axis_index("s")
    w = s * NC + c
    zero16 = jnp.zeros((16,), jnp.float32)
    one16 = jnp.ones((16,), jnp.float32)

    @pl.loop(0, 80)
    def _zero(i):
        zbuf[pl.ds(i * 16, 16)] = zero16

    @pl.loop(0, BATCH // 16)
    def _ones(i):
        ones_v[pl.ds(i * 16, 16)] = one16

    pltpu.sync_copy(zbuf, deg_sh.at[pl.ds(s * 1280, 1280)])
    pltpu.sync_copy(src_hbm.at[pl.ds(w * NBATCH, NBATCH)], idx_a)
    pltpu.sync_copy(dst_hbm.at[pl.ds(w * NBATCH, NBATCH)], idx_b)

    @pl.loop(0, NBATCH)
    def _shift(j):
        for k in range(BATCH // 16):
            sl = pl.ds(k * 16, 16)
            idx_b[j, sl] = idx_b[j, sl] + NPAD

    plsc.subcore_barrier()

    def _fire(j, sem):
        pltpu.async_copy(ones_v, deg_sh.at[idx_a.at[j]], sem, add=True)
        pltpu.async_copy(ones_v, deg_sh.at[idx_b.at[j]], sem, add=True)

    def _drain(j, sem):
        pltpu.make_async_copy(ones_v, deg_sh.at[idx_a.at[j]], sem).wait()
        pltpu.make_async_copy(ones_v, deg_sh.at[idx_b.at[j]], sem).wait()

    _fire(0, dsem0)
    _fire(1, dsem1)

    @pl.loop(0, NBATCH // 2 - 1)
    def _acc(jj):
        j = jj * 2
        _drain(j, dsem0)
        _fire(j + 2, dsem0)
        _drain(j + 1, dsem1)
        _fire(j + 3, dsem1)

    _drain(NBATCH - 2, dsem0)
    _drain(NBATCH - 1, dsem1)
    plsc.subcore_barrier()
    sl = pl.ds(s * 1280, 1280)
    pltpu.sync_copy(deg_sh.at[sl], zbuf)
    pltpu.sync_copy(zbuf, out_hbm.at[pl.ds(c * (2 * NPAD) + s * 1280, 1280)])


_deg_call = pl.kernel(
    _deg_body,
    out_type=jax.ShapeDtypeStruct((2 * 2 * NPAD,), jnp.float32),
    mesh=_mesh,
    scratch_types=[
        pltpu.VMEM((NBATCH, BATCH), jnp.int32),
        pltpu.VMEM((NBATCH, BATCH), jnp.int32),
        pltpu.VMEM((BATCH,), jnp.float32),
        pltpu.VMEM((1280,), jnp.float32),
        pltpu.SemaphoreType.DMA,
        pltpu.SemaphoreType.DMA,
        pltpu.VMEM_SHARED((2 * NPAD,), jnp.float32),
    ],
    compiler_params=_sc_params,
)


_G = 1


def _agg_loop(nbatch, h_hbm, idx_s, idx_d, rows, gsems, ssems, acc_sh):
    ngrp = nbatch // _G

    def _gather(grp, bank):
        for b in range(_G):
            pltpu.async_copy(h_hbm.at[idx_s.at[grp * _G + b]],
                             rows.at[bank * _G + b], gsems[bank])

    def _drain_gather(grp, bank):
        for b in range(_G):
            pltpu.make_async_copy(h_hbm.at[idx_s.at[grp * _G + b]],
                                  rows.at[bank * _G + b], gsems[bank]).wait()

    def _scatter(grp, bank):
        for b in range(_G):
            pltpu.async_copy(rows.at[bank * _G + b],
                             acc_sh.at[idx_d.at[grp * _G + b]], ssems[bank],
                             add=True)

    def _drain_scatter(grp, bank):
        for b in range(_G):
            pltpu.make_async_copy(rows.at[bank * _G + b],
                                  acc_sh.at[idx_d.at[grp * _G + b]],
                                  ssems[bank]).wait()

    _gather(0, 0)
    _gather(1, 1)

    @pl.loop(0, (ngrp - 2) // 2)
    def _run(jj):
        g = jj * 2
        for bank in range(2):
            _drain_gather(g + bank, bank)
            _scatter(g + bank, bank)
            _drain_scatter(g + bank, bank)
            _gather(g + 2 + bank, bank)

    for bank in range(2):
        _drain_gather(ngrp - 2 + bank, bank)
        _scatter(ngrp - 2 + bank, bank)
        _drain_scatter(ngrp - 2 + bank, bank)
    plsc.subcore_barrier()


def _make_agg_body(D, mul):
    def _body(h_hbm, src_hbm, dst_hbm, out_hbm, idx_s, idx_d, rows, stage,
              gsem0, gsem1, ssem0, ssem1, acc_sh):
        c = lax.axis_index("c")
        s = lax.axis_index("s")
        zero16 = jnp.zeros((16,), jnp.float32)

        @pl.loop(0, 128)
        def _zr(i):
            @pl.loop(0, D // 16)
            def _zc(j):
                stage[i, pl.ds(j * 16, 16)] = zero16

        for k in range(5):
            pltpu.sync_copy(stage, acc_sh.at[pl.ds(s * 640 + k * 128, 128)])
        pltpu.sync_copy(src_hbm.at[pl.ds(s * NB1, NB1)], idx_s)
        pltpu.sync_copy(dst_hbm.at[pl.ds(s * NB1, NB1)], idx_d)

        @pl.loop(0, NB1)
        def _shift(j):
            for k in range(BATCH // 16):
                sl = pl.ds(k * 16, 16)
                idx_s[j, sl] = idx_s[j, sl] * mul + c

        plsc.subcore_barrier()
        _agg_loop(NB1, h_hbm, idx_s, idx_d, rows, (gsem0, gsem1),
                  (ssem0, ssem1), acc_sh)
        for k in range(5):
            r0 = s * 640 + k * 128
            pltpu.sync_copy(acc_sh.at[pl.ds(r0, 128)], stage)
            pltpu.sync_copy(stage,
                            out_hbm.at[pl.ds(r0, 128), pl.ds(c * D, D)])

    return _body


def _make_agg(D, mul):
    return pl.kernel(
        _make_agg_body(D, mul),
        out_type=jax.ShapeDtypeStruct((NPAD, 128), jnp.float32),
        mesh=_mesh,
        scratch_types=[
            pltpu.VMEM((NB1, BATCH), jnp.int32),
            pltpu.VMEM((NB1, BATCH), jnp.int32),
            pltpu.VMEM((2 * _G, BATCH, D), jnp.float32),
            pltpu.VMEM((128, D), jnp.float32),
            pltpu.SemaphoreType.DMA,
            pltpu.SemaphoreType.DMA,
            pltpu.SemaphoreType.DMA,
            pltpu.SemaphoreType.DMA,
            pltpu.VMEM_SHARED((NPAD, D), jnp.float32),
        ],
        compiler_params=_sc_params,
    )


_agg1 = _make_agg(64, 2)
_agg2 = _make_agg(32, 4)


_BLK = 2048
_GRID = NPAD // _BLK


def _tc1_body(x_ref, w_ref, dg_ref, h_ref):
    a = lax.rsqrt(jnp.maximum(dg_ref[0, :] + dg_ref[2, :], 1.0))
    h = jnp.dot(x_ref[...], w_ref[...], preferred_element_type=jnp.float32)
    h_ref[...] = h * a[:, None]


def _tc1(x, W1, deg4):
    return pl.pallas_call(
        _tc1_body,
        grid=(_GRID,),
        in_specs=[
            pl.BlockSpec((_BLK, F), lambda i: (i, 0)),
            pl.BlockSpec((F, F), lambda i: (0, 0)),
            pl.BlockSpec((4, _BLK), lambda i: (0, i)),
        ],
        out_specs=pl.BlockSpec((_BLK, F), lambda i: (i, 0)),
        out_shape=jax.ShapeDtypeStruct((NPAD, F), jnp.float32),
    )(x, W1, deg4)


def _tc2_body(p_ref, dg_ref, b1_ref, w2_ref, out_ref):
    a = lax.rsqrt(jnp.maximum(dg_ref[0, :] + dg_ref[2, :], 1.0))
    b = lax.rsqrt(jnp.maximum(dg_ref[1, :] + dg_ref[3, :], 1.0))
    h = jnp.maximum(p_ref[...] * b[:, None] + b1_ref[0, :][None, :], 0.0)
    hw = jnp.dot(h, w2_ref[...], preferred_element_type=jnp.float32)
    hws = hw * a[:, None]
    out_ref[...] = jnp.concatenate(
        [hws, jnp.zeros((_BLK, 128 - CPAD), jnp.float32)], axis=1)


def _tc2(p, deg4, b1_2d, W2p):
    return pl.pallas_call(
        _tc2_body,
        grid=(_GRID,),
        in_specs=[
            pl.BlockSpec((_BLK, F), lambda i: (i, 0)),
            pl.BlockSpec((4, _BLK), lambda i: (0, i)),
            pl.BlockSpec((1, F), lambda i: (0, 0)),
            pl.BlockSpec((F, CPAD), lambda i: (0, 0)),
        ],
        out_specs=pl.BlockSpec((_BLK, F), lambda i: (i, 0)),
        out_shape=jax.ShapeDtypeStruct((NPAD, F), jnp.float32),
    )(p, deg4, b1_2d, W2p)


def _tc3_body(p_ref, dg_ref, b2_ref, out_ref):
    b = lax.rsqrt(jnp.maximum(dg_ref[1, :] + dg_ref[3, :], 1.0))
    out_ref[...] = p_ref[:, :C] * b[:, None] + b2_ref[0, :][None, :]


def _tc3(p, deg4, b2_2d):
    return pl.pallas_call(
        _tc3_body,
        grid=(pl.cdiv(N, _BLK),),
        in_specs=[
            pl.BlockSpec((_BLK, F), lambda i: (i, 0)),
            pl.BlockSpec((4, _BLK), lambda i: (0, i)),
            pl.BlockSpec((1, C), lambda i: (0, 0)),
        ],
        out_specs=pl.BlockSpec((_BLK, C), lambda i: (i, 0)),
        out_shape=jax.ShapeDtypeStruct((N, C), jnp.float32),
    )(p, deg4, b2_2d)


def kernel(x, edge_index, W1, b1, W2, b2):
    src = edge_index[0]
    dst = edge_index[1]
    pad_i = N + (jnp.arange(EPAD - E, dtype=jnp.int32) % PAD_ROWS)
    src_p = jnp.concatenate([src, pad_i]).reshape(ROWS2D, BATCH)
    dst_p = jnp.concatenate([dst, pad_i]).reshape(ROWS2D, BATCH)

    deg4 = _deg_call(src_p, dst_p).reshape(4, NPAD)

    W2p = jnp.pad(W2, ((0, 0), (0, CPAD - C)))
    b1_2d = b1.reshape(1, F)
    b2_2d = b2.reshape(1, C)

    h1s = _tc1(x, W1, deg4)
    agg1 = _agg1(h1s.reshape(2 * NPAD, 64), src_p, dst_p)
    h2s = _tc2(agg1, deg4, b1_2d, W2p)
    agg2 = _agg2(h2s.reshape(4 * NPAD, 32), src_p, dst_p)
    return _tc3(agg2, deg4, b2_2d)


# speedup vs baseline: 38.1264x; 1.0496x over previous
import jax
import jax.numpy as jnp
from jax import lax
from jax.experimental import pallas as pl
from jax.experimental.pallas import tpu as pltpu
from jax.experimental.pallas import tpu_sc as plsc

N = 10000
NPAD = 10240
PAD_ROWS = NPAD - N
E = 320000
NC, NS = 2, 16
NW = NC * NS
BATCH = 256
NBATCH = 40
EPAD = NW * NBATCH * BATCH
BD = 512
NBD = EPAD // (NW * BD)
ROWS2D = EPAD // BATCH
NB1 = ROWS2D // NS
F = 128
C = 40
CPAD = 64

_mesh = plsc.VectorSubcoreMesh(
    core_axis_name="c", subcore_axis_name="s", num_cores=NC, num_subcores=NS)
_sc_params = pltpu.CompilerParams(use_tc_tiling_on_sc=False)


def _deg_body(src_hbm, dst_hbm, out_hbm, idx_a, idx_b, ones_v, zbuf, dsem0,
              dsem1, deg_sh):
    c = lax.axis_index("c")
    s = lax.axis_index("s")
    w = s * NC + c
    zero16 = jnp.zeros((16,), jnp.float32)
    one16 = jnp.ones((16,), jnp.float32)

    @pl.loop(0, 80)
    def _zero(i):
        zbuf[pl.ds(i * 16, 16)] = zero16

    @pl.loop(0, BD // 16)
    def _ones(i):
        ones_v[pl.ds(i * 16, 16)] = one16

    pltpu.sync_copy(zbuf, deg_sh.at[pl.ds(s * 1280, 1280)])
    pltpu.sync_copy(src_hbm.at[pl.ds(w * NBD, NBD)], idx_a)
    pltpu.sync_copy(dst_hbm.at[pl.ds(w * NBD, NBD)], idx_b)

    @pl.loop(0, NBD)
    def _shift(j):
        for k in range(BD // 16):
            sl = pl.ds(k * 16, 16)
            idx_b[j, sl] = idx_b[j, sl] + NPAD

    plsc.subcore_barrier()

    def _fire(j, sem):
        pltpu.async_copy(ones_v, deg_sh.at[idx_a.at[j]], sem, add=True)
        pltpu.async_copy(ones_v, deg_sh.at[idx_b.at[j]], sem, add=True)

    def _drain(j, sem):
        pltpu.make_async_copy(ones_v, deg_sh.at[idx_a.at[j]], sem).wait()
        pltpu.make_async_copy(ones_v, deg_sh.at[idx_b.at[j]], sem).wait()

    _fire(0, dsem0)
    _fire(1, dsem1)

    @pl.loop(0, NBD // 2 - 1)
    def _acc(jj):
        j = jj * 2
        _drain(j, dsem0)
        _fire(j + 2, dsem0)
        _drain(j + 1, dsem1)
        _fire(j + 3, dsem1)

    _drain(NBD - 2, dsem0)
    _drain(NBD - 1, dsem1)
    plsc.subcore_barrier()
    sl = pl.ds(s * 1280, 1280)
    pltpu.sync_copy(deg_sh.at[sl], zbuf)
    pltpu.sync_copy(zbuf, out_hbm.at[pl.ds(c * (2 * NPAD) + s * 1280, 1280)])


_deg_call = pl.kernel(
    _deg_body,
    out_type=jax.ShapeDtypeStruct((2 * 2 * NPAD,), jnp.float32),
    mesh=_mesh,
    scratch_types=[
        pltpu.VMEM((NBD, BD), jnp.int32),
        pltpu.VMEM((NBD, BD), jnp.int32),
        pltpu.VMEM((BD,), jnp.float32),
        pltpu.VMEM((1280,), jnp.float32),
        pltpu.SemaphoreType.DMA,
        pltpu.SemaphoreType.DMA,
        pltpu.VMEM_SHARED((2 * NPAD,), jnp.float32),
    ],
    compiler_params=_sc_params,
)


_G = 1


def _agg_loop(nbatch, h_hbm, idx_s, idx_d, rows, gsems, ssems, acc_sh):
    ngrp = nbatch // _G

    def _gather(grp, bank):
        for b in range(_G):
            pltpu.async_copy(h_hbm.at[idx_s.at[grp * _G + b]],
                             rows.at[bank * _G + b], gsems[bank])

    def _drain_gather(grp, bank):
        for b in range(_G):
            pltpu.make_async_copy(h_hbm.at[idx_s.at[grp * _G + b]],
                                  rows.at[bank * _G + b], gsems[bank]).wait()

    def _scatter(grp, bank):
        for b in range(_G):
            pltpu.async_copy(rows.at[bank * _G + b],
                             acc_sh.at[idx_d.at[grp * _G + b]], ssems[bank],
                             add=True)

    def _drain_scatter(grp, bank):
        for b in range(_G):
            pltpu.make_async_copy(rows.at[bank * _G + b],
                                  acc_sh.at[idx_d.at[grp * _G + b]],
                                  ssems[bank]).wait()

    _gather(0, 0)
    _gather(1, 1)

    @pl.loop(0, (ngrp - 2) // 2)
    def _run(jj):
        g = jj * 2
        for bank in range(2):
            _drain_gather(g + bank, bank)
            _scatter(g + bank, bank)
            _drain_scatter(g + bank, bank)
            _gather(g + 2 + bank, bank)

    for bank in range(2):
        _drain_gather(ngrp - 2 + bank, bank)
        _scatter(ngrp - 2 + bank, bank)
        _drain_scatter(ngrp - 2 + bank, bank)
    plsc.subcore_barrier()


def _make_agg_body(D, mul, B, NB):
    def _body(h_hbm, src_hbm, dst_hbm, out_hbm, idx_s, idx_d, rows, stage,
              gsem0, gsem1, ssem0, ssem1, acc_sh):
        c = lax.axis_index("c")
        s = lax.axis_index("s")
        zero16 = jnp.zeros((16,), jnp.float32)

        @pl.loop(0, 128)
        def _zr(i):
            @pl.loop(0, D // 16)
            def _zc(j):
                stage[i, pl.ds(j * 16, 16)] = zero16

        for k in range(5):
            pltpu.sync_copy(stage, acc_sh.at[pl.ds(s * 640 + k * 128, 128)])
        pltpu.sync_copy(src_hbm.at[pl.ds(s * NB, NB)], idx_s)
        pltpu.sync_copy(dst_hbm.at[pl.ds(s * NB, NB)], idx_d)

        @pl.loop(0, NB)
        def _shift(j):
            for k in range(B // 16):
                sl = pl.ds(k * 16, 16)
                idx_s[j, sl] = idx_s[j, sl] * mul + c

        plsc.subcore_barrier()
        _agg_loop(NB, h_hbm, idx_s, idx_d, rows, (gsem0, gsem1),
                  (ssem0, ssem1), acc_sh)
        for k in range(5):
            r0 = s * 640 + k * 128
            pltpu.sync_copy(acc_sh.at[pl.ds(r0, 128)], stage)
            pltpu.sync_copy(stage,
                            out_hbm.at[pl.ds(r0, 128), pl.ds(c * D, D)])

    return _body


def _make_agg(D, mul, B, NB):
    return pl.kernel(
        _make_agg_body(D, mul, B, NB),
        out_type=jax.ShapeDtypeStruct((NPAD, 128), jnp.float32),
        mesh=_mesh,
        scratch_types=[
            pltpu.VMEM((NB, B), jnp.int32),
            pltpu.VMEM((NB, B), jnp.int32),
            pltpu.VMEM((2 * _G, B, D), jnp.float32),
            pltpu.VMEM((128, D), jnp.float32),
            pltpu.SemaphoreType.DMA,
            pltpu.SemaphoreType.DMA,
            pltpu.SemaphoreType.DMA,
            pltpu.SemaphoreType.DMA,
            pltpu.VMEM_SHARED((NPAD, D), jnp.float32),
        ],
        compiler_params=_sc_params,
    )


_agg1 = _make_agg(64, 2, BATCH, NB1)
_agg2 = _make_agg(32, 4, BD, EPAD // (NS * BD))


_BLK = 2048
_GRID = NPAD // _BLK


def _tc1_body(x_ref, w_ref, dg_ref, h_ref):
    a = lax.rsqrt(jnp.maximum(dg_ref[0, :] + dg_ref[2, :], 1.0))
    h = jnp.dot(x_ref[...], w_ref[...], preferred_element_type=jnp.float32)
    h_ref[...] = h * a[:, None]


def _tc1(x, W1, deg4):
    return pl.pallas_call(
        _tc1_body,
        grid=(_GRID,),
        in_specs=[
            pl.BlockSpec((_BLK, F), lambda i: (i, 0)),
            pl.BlockSpec((F, F), lambda i: (0, 0)),
            pl.BlockSpec((4, _BLK), lambda i: (0, i)),
        ],
        out_specs=pl.BlockSpec((_BLK, F), lambda i: (i, 0)),
        out_shape=jax.ShapeDtypeStruct((NPAD, F), jnp.float32),
    )(x, W1, deg4)


def _tc2_body(p_ref, dg_ref, b1_ref, w2_ref, out_ref):
    a = lax.rsqrt(jnp.maximum(dg_ref[0, :] + dg_ref[2, :], 1.0))
    b = lax.rsqrt(jnp.maximum(dg_ref[1, :] + dg_ref[3, :], 1.0))
    h = jnp.maximum(p_ref[...] * b[:, None] + b1_ref[0, :][None, :], 0.0)
    hw = jnp.dot(h, w2_ref[...], preferred_element_type=jnp.float32)
    hws = hw * a[:, None]
    out_ref[...] = jnp.concatenate(
        [hws, jnp.zeros((_BLK, 128 - CPAD), jnp.float32)], axis=1)


def _tc2(p, deg4, b1_2d, W2p):
    return pl.pallas_call(
        _tc2_body,
        grid=(_GRID,),
        in_specs=[
            pl.BlockSpec((_BLK, F), lambda i: (i, 0)),
            pl.BlockSpec((4, _BLK), lambda i: (0, i)),
            pl.BlockSpec((1, F), lambda i: (0, 0)),
            pl.BlockSpec((F, CPAD), lambda i: (0, 0)),
        ],
        out_specs=pl.BlockSpec((_BLK, F), lambda i: (i, 0)),
        out_shape=jax.ShapeDtypeStruct((NPAD, F), jnp.float32),
    )(p, deg4, b1_2d, W2p)


def _tc3_body(p_ref, dg_ref, b2_ref, out_ref):
    b = lax.rsqrt(jnp.maximum(dg_ref[1, :] + dg_ref[3, :], 1.0))
    out_ref[...] = p_ref[:, :C] * b[:, None] + b2_ref[0, :][None, :]


def _tc3(p, deg4, b2_2d):
    return pl.pallas_call(
        _tc3_body,
        grid=(pl.cdiv(N, _BLK),),
        in_specs=[
            pl.BlockSpec((_BLK, F), lambda i: (i, 0)),
            pl.BlockSpec((4, _BLK), lambda i: (0, i)),
            pl.BlockSpec((1, C), lambda i: (0, 0)),
        ],
        out_specs=pl.BlockSpec((_BLK, C), lambda i: (i, 0)),
        out_shape=jax.ShapeDtypeStruct((N, C), jnp.float32),
    )(p, deg4, b2_2d)


def kernel(x, edge_index, W1, b1, W2, b2):
    src = edge_index[0]
    dst = edge_index[1]
    pad_i = N + (jnp.arange(EPAD - E, dtype=jnp.int32) % PAD_ROWS)
    src_p = jnp.concatenate([src, pad_i]).reshape(ROWS2D, BATCH)
    dst_p = jnp.concatenate([dst, pad_i]).reshape(ROWS2D, BATCH)

    deg4 = _deg_call(src_p.reshape(EPAD // BD, BD),
                     dst_p.reshape(EPAD // BD, BD)).reshape(4, NPAD)

    W2p = jnp.pad(W2, ((0, 0), (0, CPAD - C)))
    b1_2d = b1.reshape(1, F)
    b2_2d = b2.reshape(1, C)

    h1s = _tc1(x, W1, deg4)
    agg1 = _agg1(h1s.reshape(2 * NPAD, 64), src_p, dst_p)
    h2s = _tc2(agg1, deg4, b1_2d, W2p)
    agg2 = _agg2(h2s.reshape(4 * NPAD, 32), src_p.reshape(EPAD // BD, BD),
                 dst_p.reshape(EPAD // BD, BD))
    return _tc3(agg2, deg4, b2_2d)


# speedup vs baseline: 38.1709x; 1.0012x over previous
import jax
import jax.numpy as jnp
from jax import lax
from jax.experimental import pallas as pl
from jax.experimental.pallas import tpu as pltpu
from jax.experimental.pallas import tpu_sc as plsc

N = 10000
NPAD = 10240
PAD_ROWS = NPAD - N
E = 320000
NC, NS = 2, 16
NW = NC * NS
BATCH = 256
NBATCH = 40
EPAD = NW * NBATCH * BATCH
BD = 512
NBD = EPAD // (NW * BD)
ROWS2D = EPAD // BATCH
NB1 = ROWS2D // NS
F = 128
C = 40
CPAD = 64

_mesh = plsc.VectorSubcoreMesh(
    core_axis_name="c", subcore_axis_name="s", num_cores=NC, num_subcores=NS)
_sc_params = pltpu.CompilerParams(use_tc_tiling_on_sc=False)


def _deg_body(src_hbm, dst_hbm, out_hbm, idx_a, idx_b, ones_v, zbuf, dsem0,
              dsem1, deg_sh):
    c = lax.axis_index("c")
    s = lax.axis_index("s")
    w = s * NC + c
    zero16 = jnp.zeros((16,), jnp.float32)
    one16 = jnp.ones((16,), jnp.float32)

    @pl.loop(0, 80)
    def _zero(i):
        zbuf[pl.ds(i * 16, 16)] = zero16

    @pl.loop(0, BD // 16)
    def _ones(i):
        ones_v[pl.ds(i * 16, 16)] = one16

    pltpu.sync_copy(zbuf, deg_sh.at[pl.ds(s * 1280, 1280)])
    pltpu.sync_copy(src_hbm.at[pl.ds(w * NBD, NBD)], idx_a)
    pltpu.sync_copy(dst_hbm.at[pl.ds(w * NBD, NBD)], idx_b)

    @pl.loop(0, NBD)
    def _shift(j):
        for k in range(BD // 16):
            sl = pl.ds(k * 16, 16)
            idx_b[j, sl] = idx_b[j, sl] + NPAD

    plsc.subcore_barrier()

    def _fire(j, sem):
        pltpu.async_copy(ones_v, deg_sh.at[idx_a.at[j]], sem, add=True)
        pltpu.async_copy(ones_v, deg_sh.at[idx_b.at[j]], sem, add=True)

    def _drain(j, sem):
        pltpu.make_async_copy(ones_v, deg_sh.at[idx_a.at[j]], sem).wait()
        pltpu.make_async_copy(ones_v, deg_sh.at[idx_b.at[j]], sem).wait()

    _fire(0, dsem0)
    _fire(1, dsem1)

    @pl.loop(0, NBD // 2 - 1)
    def _acc(jj):
        j = jj * 2
        _drain(j, dsem0)
        _fire(j + 2, dsem0)
        _drain(j + 1, dsem1)
        _fire(j + 3, dsem1)

    _drain(NBD - 2, dsem0)
    _drain(NBD - 1, dsem1)
    plsc.subcore_barrier()
    sl = pl.ds(s * 1280, 1280)
    pltpu.sync_copy(deg_sh.at[sl], zbuf)
    pltpu.sync_copy(zbuf, out_hbm.at[pl.ds(c * (2 * NPAD) + s * 1280, 1280)])


_deg_call = pl.kernel(
    _deg_body,
    out_type=jax.ShapeDtypeStruct((8 * NPAD,), jnp.float32),
    mesh=_mesh,
    scratch_types=[
        pltpu.VMEM((NBD, BD), jnp.int32),
        pltpu.VMEM((NBD, BD), jnp.int32),
        pltpu.VMEM((BD,), jnp.float32),
        pltpu.VMEM((1280,), jnp.float32),
        pltpu.SemaphoreType.DMA,
        pltpu.SemaphoreType.DMA,
        pltpu.VMEM_SHARED((2 * NPAD,), jnp.float32),
    ],
    compiler_params=_sc_params,
)


_G = 1


def _agg_loop(nbatch, h_hbm, idx_s, idx_d, rows, gsems, ssems, acc_sh):
    ngrp = nbatch // _G

    def _gather(grp, bank):
        for b in range(_G):
            pltpu.async_copy(h_hbm.at[idx_s.at[grp * _G + b]],
                             rows.at[bank * _G + b], gsems[bank])

    def _drain_gather(grp, bank):
        for b in range(_G):
            pltpu.make_async_copy(h_hbm.at[idx_s.at[grp * _G + b]],
                                  rows.at[bank * _G + b], gsems[bank]).wait()

    def _scatter(grp, bank):
        for b in range(_G):
            pltpu.async_copy(rows.at[bank * _G + b],
                             acc_sh.at[idx_d.at[grp * _G + b]], ssems[bank],
                             add=True)

    def _drain_scatter(grp, bank):
        for b in range(_G):
            pltpu.make_async_copy(rows.at[bank * _G + b],
                                  acc_sh.at[idx_d.at[grp * _G + b]],
                                  ssems[bank]).wait()

    _gather(0, 0)
    _gather(1, 1)

    @pl.loop(0, (ngrp - 2) // 2)
    def _run(jj):
        g = jj * 2
        for bank in range(2):
            _drain_gather(g + bank, bank)
            _scatter(g + bank, bank)
            _drain_scatter(g + bank, bank)
            _gather(g + 2 + bank, bank)

    for bank in range(2):
        _drain_gather(ngrp - 2 + bank, bank)
        _scatter(ngrp - 2 + bank, bank)
        _drain_scatter(ngrp - 2 + bank, bank)
    plsc.subcore_barrier()


def _make_agg_body(D, mul, B, NB):
    def _body(h_hbm, src_hbm, dst_hbm, out_hbm, idx_s, idx_d, rows, stage,
              gsem0, gsem1, ssem0, ssem1, acc_sh):
        c = lax.axis_index("c")
        s = lax.axis_index("s")
        zero16 = jnp.zeros((16,), jnp.float32)

        @pl.loop(0, 128)
        def _zr(i):
            @pl.loop(0, D // 16)
            def _zc(j):
                stage[i, pl.ds(j * 16, 16)] = zero16

        for k in range(5):
            pltpu.sync_copy(stage, acc_sh.at[pl.ds(s * 640 + k * 128, 128)])
        pltpu.sync_copy(src_hbm.at[pl.ds(s * NB, NB)], idx_s)
        pltpu.sync_copy(dst_hbm.at[pl.ds(s * NB, NB)], idx_d)

        @pl.loop(0, NB)
        def _shift(j):
            for k in range(B // 16):
                sl = pl.ds(k * 16, 16)
                idx_s[j, sl] = idx_s[j, sl] * mul + c

        plsc.subcore_barrier()
        _agg_loop(NB, h_hbm, idx_s, idx_d, rows, (gsem0, gsem1),
                  (ssem0, ssem1), acc_sh)
        for k in range(5):
            r0 = s * 640 + k * 128
            pltpu.sync_copy(acc_sh.at[pl.ds(r0, 128)], stage)
            pltpu.sync_copy(stage,
                            out_hbm.at[pl.ds(r0, 128), pl.ds(c * D, D)])

    return _body


def _make_agg(D, mul, B, NB):
    return pl.kernel(
        _make_agg_body(D, mul, B, NB),
        out_type=jax.ShapeDtypeStruct((NPAD, 128), jnp.float32),
        mesh=_mesh,
        scratch_types=[
            pltpu.VMEM((NB, B), jnp.int32),
            pltpu.VMEM((NB, B), jnp.int32),
            pltpu.VMEM((2 * _G, B, D), jnp.float32),
            pltpu.VMEM((128, D), jnp.float32),
            pltpu.SemaphoreType.DMA,
            pltpu.SemaphoreType.DMA,
            pltpu.SemaphoreType.DMA,
            pltpu.SemaphoreType.DMA,
            pltpu.VMEM_SHARED((NPAD, D), jnp.float32),
        ],
        compiler_params=_sc_params,
    )


_agg1 = _make_agg(64, 2, BATCH, NB1)
_agg2 = _make_agg(32, 4, BD, EPAD // (NS * BD))


_BLK = 2048
_GRID = NPAD // _BLK


def _tc1_body(x_ref, w_ref, dg_ref, h_ref):
    a = lax.rsqrt(jnp.maximum(dg_ref[0, :] + dg_ref[2, :], 1.0))
    h = jnp.dot(x_ref[...], w_ref[...], preferred_element_type=jnp.float32)
    h_ref[...] = h * a[:, None]


def _tc1(x, W1, deg4):
    return pl.pallas_call(
        _tc1_body,
        grid=(_GRID,),
        in_specs=[
            pl.BlockSpec((_BLK, F), lambda i: (i, 0)),
            pl.BlockSpec((F, F), lambda i: (0, 0)),
            pl.BlockSpec((8, _BLK), lambda i: (0, i)),
        ],
        out_specs=pl.BlockSpec((_BLK, F), lambda i: (i, 0)),
        out_shape=jax.ShapeDtypeStruct((NPAD, F), jnp.float32),
    )(x, W1, deg4)


def _tc2_body(p_ref, dg_ref, b1_ref, w2_ref, out_ref):
    a = lax.rsqrt(jnp.maximum(dg_ref[0, :] + dg_ref[2, :], 1.0))
    b = lax.rsqrt(jnp.maximum(dg_ref[1, :] + dg_ref[3, :], 1.0))
    h = jnp.maximum(p_ref[...] * b[:, None] + b1_ref[0, :][None, :], 0.0)
    hw = jnp.dot(h, w2_ref[...], preferred_element_type=jnp.float32)
    hws = hw * a[:, None]
    out_ref[...] = jnp.concatenate(
        [hws, jnp.zeros((_BLK, 128 - CPAD), jnp.float32)], axis=1)


def _tc2(p, deg4, b1_2d, W2p):
    return pl.pallas_call(
        _tc2_body,
        grid=(_GRID,),
        in_specs=[
            pl.BlockSpec((_BLK, F), lambda i: (i, 0)),
            pl.BlockSpec((8, _BLK), lambda i: (0, i)),
            pl.BlockSpec((1, F), lambda i: (0, 0)),
            pl.BlockSpec((F, CPAD), lambda i: (0, 0)),
        ],
        out_specs=pl.BlockSpec((_BLK, F), lambda i: (i, 0)),
        out_shape=jax.ShapeDtypeStruct((NPAD, F), jnp.float32),
    )(p, deg4, b1_2d, W2p)


def _tc3_body(p_ref, dg_ref, b2_ref, out_ref):
    b = lax.rsqrt(jnp.maximum(dg_ref[1, :] + dg_ref[3, :], 1.0))
    out_ref[...] = p_ref[:, :C] * b[:, None] + b2_ref[0, :][None, :]


def _tc3(p, deg4, b2_2d):
    return pl.pallas_call(
        _tc3_body,
        grid=(pl.cdiv(N, _BLK),),
        in_specs=[
            pl.BlockSpec((_BLK, F), lambda i: (i, 0)),
            pl.BlockSpec((8, _BLK), lambda i: (0, i)),
            pl.BlockSpec((1, C), lambda i: (0, 0)),
        ],
        out_specs=pl.BlockSpec((_BLK, C), lambda i: (i, 0)),
        out_shape=jax.ShapeDtypeStruct((N, C), jnp.float32),
    )(p, deg4, b2_2d)


def kernel(x, edge_index, W1, b1, W2, b2):
    src = edge_index[0]
    dst = edge_index[1]
    pad_i = N + (jnp.arange(EPAD - E, dtype=jnp.int32) % PAD_ROWS)
    src_p = jnp.concatenate([src, pad_i]).reshape(ROWS2D, BATCH)
    dst_p = jnp.concatenate([dst, pad_i]).reshape(ROWS2D, BATCH)

    deg4 = _deg_call(src_p.reshape(EPAD // BD, BD),
                     dst_p.reshape(EPAD // BD, BD)).reshape(8, NPAD)

    W2p = jnp.pad(W2, ((0, 0), (0, CPAD - C)))
    b1_2d = b1.reshape(1, F)
    b2_2d = b2.reshape(1, C)

    h1s = _tc1(x, W1, deg4)
    agg1 = _agg1(h1s.reshape(2 * NPAD, 64), src_p, dst_p)
    h2s = _tc2(agg1, deg4, b1_2d, W2p)
    agg2 = _agg2(h2s.reshape(4 * NPAD, 32), src_p.reshape(EPAD // BD, BD),
                 dst_p.reshape(EPAD // BD, BD))
    return _tc3(agg2, deg4, b2_2d)
